# baseline (speedup 1.0000x reference)
import functools

import jax
import jax.numpy as jnp
from jax.experimental import pallas as pl
from jax.experimental.pallas import tpu as pltpu

TOP_K = 2
NUM_EXPERTS = 8
TOKEN_TILE = 2048
EG = 2
NS = NUM_EXPERTS // EG


def _moe_dense_kernel(x_ref, wg_ref, bg_ref, we_ref, be_ref, out_ref,
                      cw_ref):
    ks = pl.program_id(1)
    x = x_ref[...]

    @pl.when(ks == 0)
    def _gating():
        scores = jnp.dot(x, wg_ref[...], preferred_element_type=jnp.float32)
        scores = scores + bg_ref[...][None, :]
        m = jnp.max(scores, axis=-1, keepdims=True)
        ex = jnp.exp(scores - m)
        probs = ex / jnp.sum(ex, axis=-1, keepdims=True)
        lane = jax.lax.broadcasted_iota(jnp.int32, probs.shape, 1)
        i1 = jnp.argmax(probs, axis=-1, keepdims=True)
        mask1 = lane == i1
        neg = jnp.where(mask1, -jnp.inf, probs)
        i2 = jnp.argmax(neg, axis=-1, keepdims=True)
        mask2 = lane == i2
        cw = jnp.where(mask1 | mask2, probs, 0.0)
        cw_ref[...] = cw
        out_ref[...] = jnp.dot(cw, be_ref[...],
                               preferred_element_type=jnp.float32)

    cw = cw_ref[...]
    xb = x.astype(jnp.bfloat16)
    acc = out_ref[...]
    for j in range(EG):
        col = jax.lax.broadcasted_iota(jnp.int32, cw.shape, 1)
        w_e = jnp.sum(jnp.where(col == ks * EG + j, cw, 0.0),
                      axis=-1, keepdims=True)
        acc = acc + w_e * jnp.dot(xb, we_ref[j],
                                  preferred_element_type=jnp.float32)
    out_ref[...] = acc


@jax.jit
def kernel(x, Wg, bg, We, be):
    T, D = x.shape
    E, _, O = We.shape
    We_b = We.astype(jnp.bfloat16)
    grid = (T // TOKEN_TILE, NS)
    return pl.pallas_call(
        _moe_dense_kernel,
        grid=grid,
        in_specs=[
            pl.BlockSpec((TOKEN_TILE, D), lambda i, ks: (i, 0)),
            pl.BlockSpec((D, E), lambda i, ks: (0, 0)),
            pl.BlockSpec((E,), lambda i, ks: (0,)),
            pl.BlockSpec((EG, D, O), lambda i, ks: (ks, 0, 0)),
            pl.BlockSpec((E, O), lambda i, ks: (0, 0)),
        ],
        out_specs=pl.BlockSpec((TOKEN_TILE, O), lambda i, ks: (i, 0)),
        out_shape=jax.ShapeDtypeStruct((T, O), jnp.float32),
        scratch_shapes=[pltpu.VMEM((TOKEN_TILE, NUM_EXPERTS), jnp.float32)],
        compiler_params=pltpu.CompilerParams(
            dimension_semantics=("arbitrary", "arbitrary"),
        ),
    )(x, Wg, bg, We_b, be)


# speedup vs baseline: 1.1170x; 1.1170x over previous
import functools

import jax
import jax.numpy as jnp
from jax.experimental import pallas as pl
from jax.experimental.pallas import tpu as pltpu

TOP_K = 2
NUM_EXPERTS = 8
TOKEN_TILE = 1024


def _moe_dense_kernel(x_ref, wg_ref, bg_ref, we_ref, be_ref, out_ref):
    x = x_ref[...]
    scores = jnp.dot(x, wg_ref[...], preferred_element_type=jnp.float32)
    scores = scores + bg_ref[...][None, :]
    m = jnp.max(scores, axis=-1, keepdims=True)
    ex = jnp.exp(scores - m)
    probs = ex / jnp.sum(ex, axis=-1, keepdims=True)
    lane = jax.lax.broadcasted_iota(jnp.int32, probs.shape, 1)
    i1 = jnp.argmax(probs, axis=-1, keepdims=True)
    mask1 = lane == i1
    neg = jnp.where(mask1, -jnp.inf, probs)
    i2 = jnp.argmax(neg, axis=-1, keepdims=True)
    mask2 = lane == i2
    cw = jnp.where(mask1 | mask2, probs, 0.0)

    xb = x.astype(jnp.bfloat16)
    acc = jnp.dot(cw, be_ref[...], preferred_element_type=jnp.float32)
    for e in range(NUM_EXPERTS):
        y = jnp.dot(xb, we_ref[e], preferred_element_type=jnp.float32)
        acc = acc + cw[:, e:e + 1] * y
    out_ref[...] = acc


@jax.jit
def kernel(x, Wg, bg, We, be):
    T, D = x.shape
    E, _, O = We.shape
    We_b = We.astype(jnp.bfloat16)
    grid = (T // TOKEN_TILE,)
    return pl.pallas_call(
        _moe_dense_kernel,
        grid=grid,
        in_specs=[
            pl.BlockSpec((TOKEN_TILE, D), lambda i: (i, 0)),
            pl.BlockSpec((D, E), lambda i: (0, 0)),
            pl.BlockSpec((E,), lambda i: (0,)),
            pl.BlockSpec((E, D, O), lambda i: (0, 0, 0)),
            pl.BlockSpec((E, O), lambda i: (0, 0)),
        ],
        out_specs=pl.BlockSpec((TOKEN_TILE, O), lambda i: (i, 0)),
        out_shape=jax.ShapeDtypeStruct((T, O), jnp.float32),
        compiler_params=pltpu.CompilerParams(
            dimension_semantics=("arbitrary",),
        ),
    )(x, Wg, bg, We_b, be)


# speedup vs baseline: 1.3432x; 1.2025x over previous
import functools

import jax
import jax.numpy as jnp
from jax.experimental import pallas as pl
from jax.experimental.pallas import tpu as pltpu

TOP_K = 2
NUM_EXPERTS = 8
TOKEN_TILE = 1024


def _moe_dense_kernel(x_ref, wg_ref, bg_ref, we_ref, be_ref, out_ref,
                      web_ref):
    i = pl.program_id(0)

    @pl.when(i == 0)
    def _cast_weights():
        for e in range(NUM_EXPERTS):
            web_ref[e] = we_ref[e].astype(jnp.bfloat16)

    x = x_ref[...]
    scores = jnp.dot(x, wg_ref[...], preferred_element_type=jnp.float32)
    scores = scores + bg_ref[...][None, :]
    m = jnp.max(scores, axis=-1, keepdims=True)
    ex = jnp.exp(scores - m)
    probs = ex / jnp.sum(ex, axis=-1, keepdims=True)
    lane = jax.lax.broadcasted_iota(jnp.int32, probs.shape, 1)
    i1 = jnp.argmax(probs, axis=-1, keepdims=True)
    mask1 = lane == i1
    neg = jnp.where(mask1, -jnp.inf, probs)
    i2 = jnp.argmax(neg, axis=-1, keepdims=True)
    mask2 = lane == i2
    cw = jnp.where(mask1 | mask2, probs, 0.0)

    xb = x.astype(jnp.bfloat16)
    acc = jnp.dot(cw, be_ref[...], preferred_element_type=jnp.float32)
    for e in range(NUM_EXPERTS):
        y = jnp.dot(xb, web_ref[e], preferred_element_type=jnp.float32)
        acc = acc + cw[:, e:e + 1] * y
    out_ref[...] = acc


@jax.jit
def kernel(x, Wg, bg, We, be):
    T, D = x.shape
    E, _, O = We.shape
    grid = (T // TOKEN_TILE,)
    return pl.pallas_call(
        _moe_dense_kernel,
        grid=grid,
        in_specs=[
            pl.BlockSpec((TOKEN_TILE, D), lambda i: (i, 0)),
            pl.BlockSpec((D, E), lambda i: (0, 0)),
            pl.BlockSpec((E,), lambda i: (0,)),
            pl.BlockSpec((E, D, O), lambda i: (0, 0, 0)),
            pl.BlockSpec((E, O), lambda i: (0, 0)),
        ],
        out_specs=pl.BlockSpec((TOKEN_TILE, O), lambda i: (i, 0)),
        out_shape=jax.ShapeDtypeStruct((T, O), jnp.float32),
        scratch_shapes=[pltpu.VMEM((E, D, O), jnp.bfloat16)],
        compiler_params=pltpu.CompilerParams(
            dimension_semantics=("arbitrary",),
        ),
    )(x, Wg, bg, We, be)
